# baseline (speedup 1.0000x reference)
import jax
import jax.numpy as jnp
from jax import lax
from jax.experimental import pallas as pl
from jax.experimental.pallas import tpu as pltpu
from jax.experimental.pallas import tpu_sc as plsc

_B, _L = 1024, 200
_V, _D, _H, _C = 1000000, 128, 512, 1000

_NC, _NS = 2, 16
_NW = _NC * _NS
_HALF = _L // 2
_LANES = 16
_DC = _D // _LANES

_NBUF = 4


def _make_pool_body(bpw):
    nh = 2 * bpw

    def _pool_body(words_hbm, emb_hbm, out_hbm, idx_v, *scratch):
        rows = scratch[:_NBUF]
        acc_v = scratch[_NBUF]
        sems = scratch[_NBUF + 1:]
        wid = lax.axis_index("s") * _NC + lax.axis_index("c")
        base = wid * bpw
        pltpu.sync_copy(words_hbm.at[pl.ds(base * 2, nh)], idx_v)

        def start(j, buf):
            pltpu.async_copy(emb_hbm.at[idx_v.at[j]], rows[buf], sems[buf])

        def wait(j, buf):
            pltpu.make_async_copy(emb_hbm.at[idx_v.at[j]], rows[buf],
                                  sems[buf]).wait()

        def reduce_half(j, buf, h):
            wait(j, buf)
            r = rows[buf]
            accs = tuple(r[0, pl.ds(c * _LANES, _LANES)] for c in range(_DC))

            def body(l, accs):
                return tuple(jnp.maximum(a, r[l, pl.ds(c * _LANES, _LANES)])
                             for c, a in enumerate(accs))

            accs = lax.fori_loop(1, _HALF, body, accs, unroll=3)
            s = j // 2
            if h == 0:
                for c in range(_DC):
                    acc_v[s, pl.ds(c * _LANES, _LANES)] = accs[c]
            else:
                for c in range(_DC):
                    acc_v[s, pl.ds(c * _LANES, _LANES)] = jnp.maximum(
                        acc_v[s, pl.ds(c * _LANES, _LANES)], accs[c])

        for j in range(_NBUF):
            start(j, j)

        @pl.loop(0, nh, step=_NBUF)
        def _(j0):
            for b in range(_NBUF):
                reduce_half(j0 + b, b, b % 2)

                @pl.when(j0 + b + _NBUF < nh)
                def _():
                    start(j0 + b + _NBUF, b)

        pltpu.sync_copy(acc_v, out_hbm.at[pl.ds(base, bpw)])

    return _pool_body


def _pool(words_chunk, emb, bc):
    bpw = bc // _NW
    mesh = plsc.VectorSubcoreMesh(core_axis_name="c", subcore_axis_name="s")
    f = pl.kernel(
        _make_pool_body(bpw),
        out_type=jax.ShapeDtypeStruct((bc, _D), jnp.float32),
        mesh=mesh,
        scratch_types=(
            [pltpu.VMEM((2 * bpw, _HALF), jnp.int32)]
            + [pltpu.VMEM((_HALF, _D), jnp.float32) for _ in range(_NBUF)]
            + [pltpu.VMEM((bpw, _D), jnp.float32)]
            + [pltpu.SemaphoreType.DMA for _ in range(_NBUF)]
        ),
    )
    return f(words_chunk, emb)


_BT = 1024


def _mlp_body(x_ref, w1_ref, w2_ref, wc_ref,
              b1_ref, g1_ref, be1_ref, m1_ref, v1_ref,
              b2_ref, g2_ref, be2_ref, m2_ref, v2_ref,
              bc_ref, gc_ref, bec_ref, mc_ref, vc_ref,
              out_ref):
    def bn(h, g, be, m, v):
        return (h - m[...]) / jnp.sqrt(v[...] + 1e-5) * g[...] + be[...]

    def elu(h):
        return jnp.where(h > 0, h, jnp.exp(h) - 1.0)

    def matmul_nt(a, w):
        return lax.dot_general(a, w, (((1,), (1,)), ((), ())),
                               preferred_element_type=jnp.float32)

    h = matmul_nt(x_ref[...], w1_ref[...])
    h = elu(bn(h + b1_ref[...], g1_ref, be1_ref, m1_ref, v1_ref))
    h = matmul_nt(h, w2_ref[...])
    h = elu(bn(h + b2_ref[...], g2_ref, be2_ref, m2_ref, v2_ref))
    o = matmul_nt(h, wc_ref[...])
    out_ref[...] = bn(o + bc_ref[...], gc_ref, bec_ref, mc_ref, vc_ref)


def _mlp(x, w1t, w2t, wct, *vecs):
    bc = x.shape[0]
    full = lambda shape: pl.BlockSpec(shape, lambda i: (0, 0))
    vec_specs = [full(v.shape) for v in vecs]
    bt = min(_BT, bc)
    return pl.pallas_call(
        _mlp_body,
        grid=(bc // bt,),
        in_specs=[pl.BlockSpec((bt, _D), lambda i: (i, 0)),
                  full((_H, _D)), full((_H, _H)), full((_C, _H))]
                 + vec_specs,
        out_specs=pl.BlockSpec((bt, _C), lambda i: (i, 0)),
        out_shape=jax.ShapeDtypeStruct((bc, _C), jnp.float32),
    )(x, w1t, w2t, wct, *vecs)


def kernel(input_words, emb, W1, b1, g1, be1, m1, v1,
           W2, b2, g2, be2, m2, v2, Wc, bc, gc, bec, mc, vc):
    words2 = input_words.astype(jnp.int32).reshape(2 * _B, _HALF)

    row = lambda v: v.reshape(1, -1)
    weights = (W1, W2, Wc,
               row(b1), row(g1), row(be1), row(m1), row(v1),
               row(b2), row(g2), row(be2), row(m2), row(v2),
               row(bc), row(gc), row(bec), row(mc), row(vc))

    return _mlp(_pool(words2, emb, _B), *weights)


# speedup vs baseline: 1.0054x; 1.0054x over previous
import jax
import jax.numpy as jnp
from jax import lax
from jax.experimental import pallas as pl
from jax.experimental.pallas import tpu as pltpu
from jax.experimental.pallas import tpu_sc as plsc

_B, _L = 1024, 200
_V, _D, _H, _C = 1000000, 128, 512, 1000

_NC, _NS = 2, 16
_NW = _NC * _NS
_HALF = _L // 2
_LANES = 16
_DC = _D // _LANES

_NBUF = 4


def _make_pool_body(bpw):
    nh = 2 * bpw

    def _pool_body(words_hbm, emb_hbm, out_hbm, idx_v, *scratch):
        rows = scratch[:_NBUF]
        acc_v = scratch[_NBUF]
        sems = scratch[_NBUF + 1:]
        wid = lax.axis_index("s") * _NC + lax.axis_index("c")
        base = wid * bpw
        pltpu.sync_copy(words_hbm.at[pl.ds(base * 2, nh)], idx_v)

        def start(j, buf):
            pltpu.async_copy(emb_hbm.at[idx_v.at[j]], rows[buf], sems[buf])

        def wait(j, buf):
            pltpu.make_async_copy(emb_hbm.at[idx_v.at[j]], rows[buf],
                                  sems[buf]).wait()

        def reduce_half(j, buf, h):
            wait(j, buf)
            r = rows[buf]
            accs = tuple(r[0, pl.ds(c * _LANES, _LANES)] for c in range(_DC))

            def body(l, accs):
                return tuple(jnp.maximum(a, r[l, pl.ds(c * _LANES, _LANES)])
                             for c, a in enumerate(accs))

            accs = lax.fori_loop(1, _HALF, body, accs, unroll=3)
            s = j // 2
            if h == 0:
                for c in range(_DC):
                    acc_v[s, pl.ds(c * _LANES, _LANES)] = accs[c]
            else:
                for c in range(_DC):
                    acc_v[s, pl.ds(c * _LANES, _LANES)] = jnp.maximum(
                        acc_v[s, pl.ds(c * _LANES, _LANES)], accs[c])

        for j in range(_NBUF):
            start(j, j)

        @pl.loop(0, nh, step=_NBUF)
        def _(j0):
            for b in range(_NBUF):
                reduce_half(j0 + b, b, b % 2)

                @pl.when(j0 + b + _NBUF < nh)
                def _():
                    start(j0 + b + _NBUF, b)

        pltpu.sync_copy(acc_v, out_hbm.at[pl.ds(base, bpw)])

    return _pool_body


def _pool(words_chunk, emb, bc):
    bpw = bc // _NW
    mesh = plsc.VectorSubcoreMesh(core_axis_name="c", subcore_axis_name="s")
    f = pl.kernel(
        _make_pool_body(bpw),
        out_type=jax.ShapeDtypeStruct((bc, _D), jnp.float32),
        mesh=mesh,
        scratch_types=(
            [pltpu.VMEM((2 * bpw, _HALF), jnp.int32)]
            + [pltpu.VMEM((_HALF, _D), jnp.float32) for _ in range(_NBUF)]
            + [pltpu.VMEM((bpw, _D), jnp.float32)]
            + [pltpu.SemaphoreType.DMA for _ in range(_NBUF)]
        ),
    )
    return f(words_chunk, emb)


_BT = 512


def _mlp_body(x_ref, w1_ref, w2_ref, wc_ref,
              b1_ref, g1_ref, be1_ref, m1_ref, v1_ref,
              b2_ref, g2_ref, be2_ref, m2_ref, v2_ref,
              bc_ref, gc_ref, bec_ref, mc_ref, vc_ref,
              out_ref):
    def bn(h, g, be, m, v):
        return (h - m[...]) / jnp.sqrt(v[...] + 1e-5) * g[...] + be[...]

    def elu(h):
        return jnp.where(h > 0, h, jnp.exp(h) - 1.0)

    def matmul_nt(a, w):
        return lax.dot_general(a, w, (((1,), (1,)), ((), ())),
                               preferred_element_type=jnp.float32)

    h = matmul_nt(x_ref[...], w1_ref[...])
    h = elu(bn(h + b1_ref[...], g1_ref, be1_ref, m1_ref, v1_ref))
    h = matmul_nt(h, w2_ref[...])
    h = elu(bn(h + b2_ref[...], g2_ref, be2_ref, m2_ref, v2_ref))
    o = matmul_nt(h, wc_ref[...])
    out_ref[...] = bn(o + bc_ref[...], gc_ref, bec_ref, mc_ref, vc_ref)


def _mlp(x, w1t, w2t, wct, *vecs):
    bc = x.shape[0]
    full = lambda shape: pl.BlockSpec(shape, lambda i: (0, 0))
    vec_specs = [full(v.shape) for v in vecs]
    bt = min(_BT, bc)
    return pl.pallas_call(
        _mlp_body,
        grid=(bc // bt,),
        in_specs=[pl.BlockSpec((bt, _D), lambda i: (i, 0)),
                  full((_H, _D)), full((_H, _H)), full((_C, _H))]
                 + vec_specs,
        out_specs=pl.BlockSpec((bt, _C), lambda i: (i, 0)),
        out_shape=jax.ShapeDtypeStruct((bc, _C), jnp.float32),
    )(x, w1t, w2t, wct, *vecs)


def kernel(input_words, emb, W1, b1, g1, be1, m1, v1,
           W2, b2, g2, be2, m2, v2, Wc, bc, gc, bec, mc, vc):
    words2 = input_words.astype(jnp.int32).reshape(2 * _B, _HALF)

    row = lambda v: v.reshape(1, -1)
    weights = (W1, W2, Wc,
               row(b1), row(g1), row(be1), row(m1), row(v1),
               row(b2), row(g2), row(be2), row(m2), row(v2),
               row(bc), row(gc), row(bec), row(mc), row(vc))

    return _mlp(_pool(words2, emb, _B), *weights)
